# baseline (speedup 1.0000x reference)
import functools

import jax
import jax.numpy as jnp
from jax import lax
from jax.experimental import pallas as pl
from jax.experimental.pallas import tpu as pltpu
from jax.experimental.pallas import tpu_sc as plsc

B = 16
N1 = 8192
N2 = 2048
C1 = 128
C2 = 256
COUT = C2 + C1

QB = 512

NC = 2
NS = 16
NW = NC * NS
TOT = B * N1
QPW = TOT // NW
G = 32
NCH = QPW // G


def _tc_body(x1_ref, x2t_ref, idx_ref, w_ref):
    b = pl.program_id(0)
    x1 = x1_ref[0]
    x2t = x2t_ref[0]
    d0 = x1[:, 0:1] - x2t[0:1, :]
    d1 = x1[:, 1:2] - x2t[1:2, :]
    d2 = x1[:, 2:3] - x2t[2:3, :]
    sqd = d0 * d0 + d1 * d1 + d2 * d2
    iota_f = lax.broadcasted_iota(jnp.int32, (QB, N2), 1).astype(jnp.float32)
    cur = sqd
    idxs, dists = [], []
    for _ in range(3):
        m = jnp.min(cur, axis=1, keepdims=True)
        i = jnp.min(jnp.where(cur == m, iota_f, 65536.0), axis=1,
                    keepdims=True)
        cur = jnp.where(iota_f == i, jnp.inf, cur)
        idxs.append(i)
        dists.append(m)
    d = jnp.concatenate(dists, axis=1)
    d = jnp.maximum(d, 1e-10)
    r = 1.0 / d
    w = r / jnp.sum(r, axis=1, keepdims=True)
    idx = jnp.concatenate(idxs, axis=1).astype(jnp.int32) + b * N2
    idx_ref[0] = idx
    w_ref[0] = w


@jax.jit
def _three_nn(xyz1, x2t):
    return pl.pallas_call(
        _tc_body,
        grid=(B, N1 // QB),
        in_specs=[
            pl.BlockSpec((1, QB, 3), lambda b, q: (b, q, 0)),
            pl.BlockSpec((1, 3, N2), lambda b, q: (b, 0, 0)),
        ],
        out_specs=[
            pl.BlockSpec((1, QB, 3), lambda b, q: (b, q, 0)),
            pl.BlockSpec((1, QB, 3), lambda b, q: (b, q, 0)),
        ],
        out_shape=[
            jax.ShapeDtypeStruct((B, N1, 3), jnp.int32),
            jax.ShapeDtypeStruct((B, N1, 3), jnp.float32),
        ],
    )(xyz1, x2t)


def _sc_body(table_hbm, idx_hbm, w_hbm, p1_hbm, out_hbm,
             idx_v, w_v, g0, g1, o0, o1,
             sem_g0, sem_g1, sem_o0, sem_o1, sem_p1):
    wid = lax.axis_index("s") * NC + lax.axis_index("c")
    q0 = wid * QPW
    gath = (g0, g1)
    ovs = (o0, o1)
    sems_g = (sem_g0, sem_g1)
    sems_o = (sem_o0, sem_o1)

    pltpu.sync_copy(idx_hbm.at[pl.ds(q0 * 3, QPW * 3)], idx_v)
    pltpu.sync_copy(w_hbm.at[pl.ds(q0 * 3, QPW * 3)],
                    w_v.at[pl.ds(0, QPW * 3)])
    pltpu.async_copy(p1_hbm.at[pl.ds(q0, QPW), :],
                     out_hbm.at[pl.ds(q0, QPW), pl.ds(C2, C1)], sem_p1)

    def fire(c, buf):
        off = pl.multiple_of(c * (3 * G), 8)
        pltpu.async_copy(table_hbm.at[idx_v.at[pl.ds(off, 3 * G)]],
                         gath[buf], sems_g[buf])

    def wait_g(buf):
        pltpu.make_async_copy(table_hbm.at[idx_v.at[pl.ds(0, 3 * G)]],
                              gath[buf], sems_g[buf]).wait()

    def store(c, buf):
        pltpu.async_copy(ovs[buf],
                         out_hbm.at[pl.ds(q0 + c * G, G), pl.ds(0, C2)],
                         sems_o[buf])

    def wait_o(buf):
        pltpu.make_async_copy(ovs[buf],
                              out_hbm.at[pl.ds(q0, G), pl.ds(0, C2)],
                              sems_o[buf]).wait()

    def compute(c, buf):
        gv = gath[buf]
        ov = ovs[buf]

        def q_body(qi, _):
            wv = w_v[pl.ds(c * (3 * G) + 3 * qi, 16)]
            w0 = wv[0]
            w1 = wv[1]
            w2 = wv[2]
            row = 3 * qi
            for j in range(C2 // 16):
                s = pl.ds(j * 16, 16)
                ov[qi, s] = (gv[row, s] * w0 + gv[row + 1, s] * w1
                             + gv[row + 2, s] * w2)
            return 0

        lax.fori_loop(0, G, q_body, 0)

    fire(0, 0)
    fire(1, 1)

    def loop_body(ci, _):
        for par in range(2):
            c = 2 * ci + par
            wait_g(par)

            @pl.when(c >= 2)
            def _():
                wait_o(par)

            compute(c, par)
            store(c, par)

            @pl.when(c + 2 < NCH)
            def _():
                fire(c + 2, par)
        return 0

    lax.fori_loop(0, NCH // 2, loop_body, 0)
    wait_o(0)
    wait_o(1)
    pltpu.make_async_copy(p1_hbm.at[pl.ds(q0, QPW), :],
                          out_hbm.at[pl.ds(q0, QPW), pl.ds(C2, C1)],
                          sem_p1).wait()


@jax.jit
def _interp(table, idx_flat, w_flat, p1_flat):
    mesh = plsc.VectorSubcoreMesh(core_axis_name="c", subcore_axis_name="s")
    f = functools.partial(
        pl.kernel,
        out_type=jax.ShapeDtypeStruct((TOT, COUT), jnp.float32),
        mesh=mesh,
        scratch_types=[
            pltpu.VMEM((QPW * 3,), jnp.int32),
            pltpu.VMEM((QPW * 3 + 16,), jnp.float32),
            pltpu.VMEM((3 * G, C2), jnp.float32),
            pltpu.VMEM((3 * G, C2), jnp.float32),
            pltpu.VMEM((G, C2), jnp.float32),
            pltpu.VMEM((G, C2), jnp.float32),
            pltpu.SemaphoreType.DMA,
            pltpu.SemaphoreType.DMA,
            pltpu.SemaphoreType.DMA,
            pltpu.SemaphoreType.DMA,
            pltpu.SemaphoreType.DMA,
        ],
    )(_sc_body)
    return f(table, idx_flat, w_flat, p1_flat)


def kernel(xyz1, xyz2, points1, points2):
    x2t = jnp.transpose(xyz2, (0, 2, 1))
    idx, w = _three_nn(xyz1, x2t)
    table = points2.reshape(B * N2, C2)
    out = _interp(table, idx.reshape(-1), w.reshape(-1),
                  points1.reshape(TOT, C1))
    return out.reshape(B, N1, 1, COUT)


# speedup vs baseline: 1.0368x; 1.0368x over previous
import functools

import jax
import jax.numpy as jnp
from jax import lax
from jax.experimental import pallas as pl
from jax.experimental.pallas import tpu as pltpu
from jax.experimental.pallas import tpu_sc as plsc

B = 16
N1 = 8192
N2 = 2048
C1 = 128
C2 = 256
COUT = C2 + C1

QB = 1024

NC = 2
NS = 16
NW = NC * NS
TOT = B * N1
QPW = TOT // NW
G = 32
NCH = QPW // G


def _tc_body(x1_ref, x2t_ref, i0_ref, i1_ref, i2_ref,
             w0_ref, w1_ref, w2_ref):
    b = pl.program_id(0)
    x1 = x1_ref[0]
    x2t = x2t_ref[0]
    d0 = x1[:, 0:1] - x2t[0:1, :]
    d1 = x1[:, 1:2] - x2t[1:2, :]
    d2 = x1[:, 2:3] - x2t[2:3, :]
    sqd = d0 * d0 + d1 * d1 + d2 * d2
    iota_f = lax.broadcasted_iota(jnp.int32, (QB, N2), 1).astype(jnp.float32)
    cur = sqd
    idxs, dists = [], []
    for _ in range(3):
        m = jnp.min(cur, axis=1, keepdims=True)
        i = jnp.min(jnp.where(cur == m, iota_f, 65536.0), axis=1,
                    keepdims=True)
        cur = jnp.where(iota_f == i, jnp.inf, cur)
        idxs.append(i)
        dists.append(m)
    d = jnp.concatenate(dists, axis=1)
    d = jnp.maximum(d, 1e-10)
    r = 1.0 / d
    w = r / jnp.sum(r, axis=1, keepdims=True)
    base = b * N2
    for k, ref in enumerate((i0_ref, i1_ref, i2_ref)):
        ref[...] = jnp.reshape(idxs[k].astype(jnp.int32) + base,
                               (QB // 128, 128))
    for k, ref in enumerate((w0_ref, w1_ref, w2_ref)):
        ref[...] = jnp.reshape(w[:, k:k + 1], (QB // 128, 128))


@jax.jit
def _three_nn(xyz1, x2t):
    nrow = QB // 128
    plane = pl.BlockSpec((nrow, 128), lambda b, q: (b * (N1 // QB) + q, 0))
    return pl.pallas_call(
        _tc_body,
        grid=(B, N1 // QB),
        in_specs=[
            pl.BlockSpec((1, QB, 3), lambda b, q: (b, q, 0)),
            pl.BlockSpec((1, 3, N2), lambda b, q: (b, 0, 0)),
        ],
        out_specs=[plane] * 6,
        out_shape=[jax.ShapeDtypeStruct((TOT // 128, 128), jnp.int32)] * 3
        + [jax.ShapeDtypeStruct((TOT // 128, 128), jnp.float32)] * 3,
    )(xyz1, x2t)


def _sc_body(table_hbm, i0h, i1h, i2h, w0h, w1h, w2h, p1_hbm, out_hbm,
             ib00, ib10, ib20, ib01, ib11, ib21,
             g00, g10, g20, g01, g11, g21,
             o0, o1, wv0, wv1, wv2,
             sem_i0, sem_i1, sem_g0, sem_g1, sem_o0, sem_o1, sem_p1):
    wid = lax.axis_index("s") * NC + lax.axis_index("c")
    q0 = wid * QPW

    ihs = (i0h, i1h, i2h)
    ibufs = ((ib00, ib10, ib20), (ib01, ib11, ib21))
    gbufs = ((g00, g10, g20), (g01, g11, g21))
    obufs = (o0, o1)
    wvs = (wv0, wv1, wv2)
    sems_i = (sem_i0, sem_i1)
    sems_g = (sem_g0, sem_g1)
    sems_o = (sem_o0, sem_o1)

    for k in range(3):
        pltpu.sync_copy((w0h, w1h, w2h)[k].at[pl.ds(q0, QPW)],
                        wvs[k].at[pl.ds(0, QPW)])
    pltpu.async_copy(p1_hbm.at[pl.ds(q0, QPW), :],
                     out_hbm.at[pl.ds(q0, QPW), pl.ds(C2, C1)], sem_p1)

    def fire_idx(c, p):
        for k in range(3):
            pltpu.async_copy(ihs[k].at[pl.ds(q0 + c * G, G)],
                             ibufs[p][k], sems_i[p])

    def wait_idx(p):
        for k in range(3):
            pltpu.make_async_copy(ihs[k].at[pl.ds(0, G)],
                                  ibufs[p][k], sems_i[p]).wait()

    def fire_gather(p):
        for k in range(3):
            pltpu.async_copy(table_hbm.at[ibufs[p][k]], gbufs[p][k],
                             sems_g[p])

    def wait_gather(p):
        for k in range(3):
            pltpu.make_async_copy(table_hbm.at[ibufs[p][k]], gbufs[p][k],
                                  sems_g[p]).wait()

    def fire_store(c, p):
        pltpu.async_copy(obufs[p],
                         out_hbm.at[pl.ds(q0 + c * G, G), pl.ds(0, C2)],
                         sems_o[p])

    def wait_store(p):
        pltpu.make_async_copy(obufs[p],
                              out_hbm.at[pl.ds(q0, G), pl.ds(0, C2)],
                              sems_o[p]).wait()

    def compute(c, p):
        g0v, g1v, g2v = gbufs[p]
        ov = obufs[p]

        def q_body(qi, _):
            off = c * G + qi
            w0 = wv0[pl.ds(off, 16)][0]
            w1 = wv1[pl.ds(off, 16)][0]
            w2 = wv2[pl.ds(off, 16)][0]
            for j in range(C2 // 16):
                s = pl.ds(j * 16, 16)
                ov[qi, s] = (g0v[qi, s] * w0 + g1v[qi, s] * w1
                             + g2v[qi, s] * w2)
            return 0

        lax.fori_loop(0, G, q_body, 0)

    fire_idx(0, 0)
    fire_idx(1, 1)
    wait_idx(0)
    fire_gather(0)

    def loop_body(ci, _):
        for p in range(2):
            c = 2 * ci + p
            op = 1 - p

            @pl.when(c + 1 < NCH)
            def _():
                wait_idx(op)
                fire_gather(op)

            wait_gather(p)

            @pl.when(c >= 2)
            def _():
                wait_store(p)

            compute(c, p)
            fire_store(c, p)

            @pl.when(c + 2 < NCH)
            def _():
                fire_idx(c + 2, p)
        return 0

    lax.fori_loop(0, NCH // 2, loop_body, 0)
    wait_store(0)
    wait_store(1)
    pltpu.make_async_copy(p1_hbm.at[pl.ds(q0, QPW), :],
                          out_hbm.at[pl.ds(q0, QPW), pl.ds(C2, C1)],
                          sem_p1).wait()


@jax.jit
def _interp(table, i0, i1, i2, w0, w1, w2, p1_flat):
    mesh = plsc.VectorSubcoreMesh(core_axis_name="c", subcore_axis_name="s")
    ib = pltpu.VMEM((G,), jnp.int32)
    gb = pltpu.VMEM((G, C2), jnp.float32)
    ob = pltpu.VMEM((G, C2), jnp.float32)
    wb = pltpu.VMEM((QPW + 16,), jnp.float32)
    f = functools.partial(
        pl.kernel,
        out_type=jax.ShapeDtypeStruct((TOT, COUT), jnp.float32),
        mesh=mesh,
        scratch_types=[ib] * 6 + [gb] * 6 + [ob] * 2 + [wb] * 3
        + [pltpu.SemaphoreType.DMA] * 7,
    )(_sc_body)
    return f(table, i0, i1, i2, w0, w1, w2, p1_flat)


def kernel(xyz1, xyz2, points1, points2):
    x2t = jnp.transpose(xyz2, (0, 2, 1))
    i0, i1, i2, w0, w1, w2 = _three_nn(xyz1, x2t)
    table = points2.reshape(B * N2, C2)
    out = _interp(table, i0.reshape(TOT), i1.reshape(TOT), i2.reshape(TOT),
                  w0.reshape(TOT), w1.reshape(TOT), w2.reshape(TOT),
                  points1.reshape(TOT, C1))
    return out.reshape(B, N1, 1, COUT)


# speedup vs baseline: 2.5350x; 2.4451x over previous
import jax
import jax.numpy as jnp
from jax import lax
from jax.experimental import pallas as pl

B = 16
N1 = 8192
N2 = 2048
C1 = 128
C2 = 256
COUT = C2 + C1

QB = 512


def _body(x1_ref, x2t_ref, p2_ref, p1_ref, out_ref):
    x1 = x1_ref[0]
    x2t = x2t_ref[0]
    d0 = x1[:, 0:1] - x2t[0:1, :]
    d1 = x1[:, 1:2] - x2t[1:2, :]
    d2 = x1[:, 2:3] - x2t[2:3, :]
    sqd = d0 * d0 + d1 * d1 + d2 * d2
    iota_f = lax.broadcasted_iota(jnp.int32, (QB, N2), 1).astype(jnp.float32)
    cur = sqd
    idxs, dists = [], []
    for _ in range(3):
        m = jnp.min(cur, axis=1, keepdims=True)
        i = jnp.min(jnp.where(cur == m, iota_f, 65536.0), axis=1,
                    keepdims=True)
        cur = jnp.where(iota_f == i, jnp.inf, cur)
        idxs.append(i)
        dists.append(m)
    d = jnp.concatenate(dists, axis=1)
    d = jnp.maximum(d, 1e-10)
    r = 1.0 / d
    w = r / jnp.sum(r, axis=1, keepdims=True)

    wmat = jnp.zeros((QB, N2), jnp.float32)
    for k in range(3):
        wmat = jnp.where(iota_f == idxs[k], w[:, k:k + 1], wmat)
    interp = lax.dot_general(
        wmat.astype(jnp.bfloat16), p2_ref[0],
        (((1,), (0,)), ((), ())),
        preferred_element_type=jnp.float32)
    out_ref[0, :, 0, 0:C2] = interp
    out_ref[0, :, 0, C2:COUT] = p1_ref[0]


@jax.jit
def _fused(xyz1, x2t, p2b, points1):
    return pl.pallas_call(
        _body,
        grid=(B, N1 // QB),
        in_specs=[
            pl.BlockSpec((1, QB, 3), lambda b, q: (b, q, 0)),
            pl.BlockSpec((1, 3, N2), lambda b, q: (b, 0, 0)),
            pl.BlockSpec((1, N2, C2), lambda b, q: (b, 0, 0)),
            pl.BlockSpec((1, QB, C1), lambda b, q: (b, q, 0)),
        ],
        out_specs=pl.BlockSpec((1, QB, 1, COUT), lambda b, q: (b, q, 0, 0)),
        out_shape=jax.ShapeDtypeStruct((B, N1, 1, COUT), jnp.float32),
    )(xyz1, x2t, p2b, points1)


def kernel(xyz1, xyz2, points1, points2):
    x2t = jnp.transpose(xyz2, (0, 2, 1))
    p2b = points2.astype(jnp.bfloat16)
    return _fused(xyz1, x2t, p2b, points1)


# speedup vs baseline: 2.6101x; 1.0296x over previous
import jax
import jax.numpy as jnp
from jax import lax
from jax.experimental import pallas as pl

B = 16
N1 = 8192
N2 = 2048
C1 = 128
C2 = 256
COUT = C2 + C1

QB = 1024


def _body(x1_ref, x2t_ref, p2_ref, p1_ref, out_ref):
    x1 = x1_ref[0]
    x2t = x2t_ref[0]
    d0 = x1[:, 0:1] - x2t[0:1, :]
    d1 = x1[:, 1:2] - x2t[1:2, :]
    d2 = x1[:, 2:3] - x2t[2:3, :]
    sqd = d0 * d0 + d1 * d1 + d2 * d2
    iota_f = lax.broadcasted_iota(jnp.int32, (QB, N2), 1).astype(jnp.float32)
    cur = sqd
    idxs, dists = [], []
    for _ in range(3):
        m = jnp.min(cur, axis=1, keepdims=True)
        i = jnp.min(jnp.where(cur == m, iota_f, 65536.0), axis=1,
                    keepdims=True)
        cur = jnp.where(iota_f == i, jnp.inf, cur)
        idxs.append(i)
        dists.append(m)
    d = jnp.concatenate(dists, axis=1)
    d = jnp.maximum(d, 1e-10)
    r = 1.0 / d
    w = r / jnp.sum(r, axis=1, keepdims=True)

    wmat = jnp.zeros((QB, N2), jnp.float32)
    for k in range(3):
        wmat = jnp.where(iota_f == idxs[k], w[:, k:k + 1], wmat)
    interp = lax.dot_general(
        wmat.astype(jnp.bfloat16), p2_ref[0],
        (((1,), (0,)), ((), ())),
        preferred_element_type=jnp.float32)
    out_ref[0, :, 0, 0:C2] = interp
    out_ref[0, :, 0, C2:COUT] = p1_ref[0]


@jax.jit
def _fused(xyz1, x2t, p2b, points1):
    return pl.pallas_call(
        _body,
        grid=(B, N1 // QB),
        in_specs=[
            pl.BlockSpec((1, QB, 3), lambda b, q: (b, q, 0)),
            pl.BlockSpec((1, 3, N2), lambda b, q: (b, 0, 0)),
            pl.BlockSpec((1, N2, C2), lambda b, q: (b, 0, 0)),
            pl.BlockSpec((1, QB, C1), lambda b, q: (b, q, 0)),
        ],
        out_specs=pl.BlockSpec((1, QB, 1, COUT), lambda b, q: (b, q, 0, 0)),
        out_shape=jax.ShapeDtypeStruct((B, N1, 1, COUT), jnp.float32),
    )(xyz1, x2t, p2b, points1)


def kernel(xyz1, xyz2, points1, points2):
    x2t = jnp.transpose(xyz2, (0, 2, 1))
    p2b = points2.astype(jnp.bfloat16)
    return _fused(xyz1, x2t, p2b, points1)
